# baseline (speedup 1.0000x reference)
import functools
import jax
import jax.numpy as jnp
import numpy as np
from jax import lax
from jax.experimental import pallas as pl
from jax.experimental.pallas import tpu as pltpu, tpu_sc as plsc

_NPOINT = 512
_RADIUS_LIST = [0.1, 0.2, 0.4]
_NSAMPLE_LIST = [16, 32, 128]
_BN_EPS = 1e-3
_CIN_PAD = 32
_TBL_W = 128


_FPS_R, _FPS_L = 64, 128


def _fps_pallas_body(B, x_ref, y_ref, z_ref, idx_ref, cx_ref, cy_ref, cz_ref):
    Xs = [x_ref[b] for b in range(B)]
    Ys = [y_ref[b] for b in range(B)]
    Zs = [z_ref[b] for b in range(B)]
    lin = (
        jax.lax.broadcasted_iota(jnp.int32, (_FPS_R, _FPS_L), 0) * _FPS_L
        + jax.lax.broadcasted_iota(jnp.int32, (_FPS_R, _FPS_L), 1)
    )
    lin_s = (
        jax.lax.broadcasted_iota(jnp.int32, (4, 128), 0) * 128
        + jax.lax.broadcasted_iota(jnp.int32, (4, 128), 1)
    )
    N = _FPS_R * _FPS_L

    def body(i, carry):
        dist, far, idxacc, cxa, cya, cza = carry
        out = ([], [], [], [], [], [])
        rec = lin_s == i
        for b in range(B):
            selm = lin == far[b]
            cx = jnp.sum(jnp.where(selm, Xs[b], 0.0), keepdims=True)
            cy = jnp.sum(jnp.where(selm, Ys[b], 0.0), keepdims=True)
            cz = jnp.sum(jnp.where(selm, Zs[b], 0.0), keepdims=True)
            d = (Xs[b] - cx) ** 2 + (Ys[b] - cy) ** 2 + (Zs[b] - cz) ** 2
            db = jnp.minimum(dist[b], d)
            m = jnp.max(db, keepdims=True)
            far2 = jnp.min(jnp.where(db == m, lin, N), keepdims=True)
            out[0].append(db)
            out[1].append(far2)
            out[2].append(jnp.where(rec, far[b], idxacc[b]))
            out[3].append(jnp.where(rec, cx, cxa[b]))
            out[4].append(jnp.where(rec, cy, cya[b]))
            out[5].append(jnp.where(rec, cz, cza[b]))
        return out

    init = (
        [jnp.full((_FPS_R, _FPS_L), 1e10, jnp.float32)] * B,
        [jnp.zeros((1, 1), jnp.int32)] * B,
        [jnp.zeros((4, 128), jnp.int32)] * B,
        [jnp.zeros((4, 128), jnp.float32)] * B,
        [jnp.zeros((4, 128), jnp.float32)] * B,
        [jnp.zeros((4, 128), jnp.float32)] * B,
    )
    _, _, idxacc, cxa, cya, cza = jax.lax.fori_loop(0, _NPOINT, body, init)
    for b in range(B):
        idx_ref[b] = idxacc[b]
        cx_ref[b] = cxa[b]
        cy_ref[b] = cya[b]
        cz_ref[b] = cza[b]


def _fps(xyz, npoint):
    B, N, _ = xyz.shape
    X = xyz[..., 0].reshape(B, _FPS_R, _FPS_L)
    Y = xyz[..., 1].reshape(B, _FPS_R, _FPS_L)
    Z = xyz[..., 2].reshape(B, _FPS_R, _FPS_L)
    blk = pl.BlockSpec((B, _FPS_R, _FPS_L), lambda: (0, 0, 0))
    oblk = pl.BlockSpec((B, 4, 128), lambda: (0, 0, 0))
    idx, cx, cy, cz = pl.pallas_call(
        functools.partial(_fps_pallas_body, B),
        grid=(),
        in_specs=[blk, blk, blk],
        out_specs=[oblk, oblk, oblk, oblk],
        out_shape=[
            jax.ShapeDtypeStruct((B, 4, 128), jnp.int32),
            jax.ShapeDtypeStruct((B, 4, 128), jnp.float32),
            jax.ShapeDtypeStruct((B, 4, 128), jnp.float32),
            jax.ShapeDtypeStruct((B, 4, 128), jnp.float32),
        ],
    )(X, Y, Z)
    fps_idx = idx.reshape(B, npoint)
    new_xyz = jnp.stack(
        [cx.reshape(B, npoint), cy.reshape(B, npoint), cz.reshape(B, npoint)], axis=-1
    )
    return fps_idx, new_xyz


_BQ_SBLK = 64


def _bq_body(nxp_ref, xyzt_ref, o1_ref, o2_ref, o3_ref):
    nxp = nxp_ref[0]
    xyzt = xyzt_ref[0]
    N = xyzt.shape[1]
    d2 = (
        jnp.sum(nxp * nxp, axis=1, keepdims=True)
        + jnp.sum(xyzt * xyzt, axis=0, keepdims=True)
        - 2.0 * jnp.dot(nxp, xyzt, preferred_element_type=jnp.float32)
    )
    linj = jax.lax.broadcasted_iota(jnp.int32, (_BQ_SBLK, N), 1).astype(jnp.float32)
    CW = 1024
    for radius, K, o_ref in (
        (_RADIUS_LIST[0], _NSAMPLE_LIST[0], o1_ref),
        (_RADIUS_LIST[1], _NSAMPLE_LIST[1], o2_ref),
        (_RADIUS_LIST[2], _NSAMPLE_LIST[2], o3_ref),
    ):
        c = jnp.where(d2 <= radius * radius, 1.0, 0.0)
        sh = 1
        while sh < N:
            c = c + jnp.concatenate(
                [jnp.zeros((_BQ_SBLK, sh), jnp.float32), c[:, : N - sh]], axis=1
            )
            sh *= 2
        cntK = jnp.sum(jnp.where(c <= jnp.float32(K - 1), 1.0, 0.0), axis=1)
        bound = jnp.max(cntK)
        kio = jax.lax.broadcasted_iota(jnp.int32, (_BQ_SBLK, K), 1)
        acc = jnp.full((_BQ_SBLK, K), jnp.float32(N), jnp.float32)

        for q in range(N // CW):
            cq = jax.lax.slice_in_dim(c, q * CW, (q + 1) * CW, axis=1)
            lq = jax.lax.slice_in_dim(linj, q * CW, (q + 1) * CW, axis=1)

            def _chunk(acc=acc, cq=cq, lq=lq):
                a = acc
                for k in range(K):
                    cand = jnp.where(cq == jnp.float32(k + 1), lq, jnp.float32(N))
                    idxk = jnp.min(cand, axis=1)
                    a = jnp.where(kio == k, jnp.minimum(a, idxk[:, None]), a)
                return a

            acc = jax.lax.cond(jnp.float32(q * CW) <= bound, _chunk, lambda acc=acc: acc)
        acc = acc.astype(jnp.int32)
        acc = jnp.where(acc == N, acc[:, :1], acc)
        o_ref[0] = acc


def _ball_query_all(xyz, new_xyz):
    B, N, _ = xyz.shape
    S = new_xyz.shape[1]
    nxp = jnp.concatenate([new_xyz, jnp.zeros((B, S, 5), jnp.float32)], axis=-1)
    xyzt = jnp.concatenate(
        [jnp.swapaxes(xyz, 1, 2), jnp.zeros((B, 5, N), jnp.float32)], axis=1
    )
    outs = pl.pallas_call(
        _bq_body,
        grid=(B, S // _BQ_SBLK),
        in_specs=[
            pl.BlockSpec((1, _BQ_SBLK, 8), lambda b, s: (b, s, 0)),
            pl.BlockSpec((1, 8, N), lambda b, s: (b, 0, 0)),
        ],
        out_specs=[
            pl.BlockSpec((1, _BQ_SBLK, _NSAMPLE_LIST[0]), lambda b, s: (b, s, 0)),
            pl.BlockSpec((1, _BQ_SBLK, _NSAMPLE_LIST[1]), lambda b, s: (b, s, 0)),
            pl.BlockSpec((1, _BQ_SBLK, _NSAMPLE_LIST[2]), lambda b, s: (b, s, 0)),
        ],
        out_shape=[
            jax.ShapeDtypeStruct((B, S, _NSAMPLE_LIST[0]), jnp.int32),
            jax.ShapeDtypeStruct((B, S, _NSAMPLE_LIST[1]), jnp.int32),
            jax.ShapeDtypeStruct((B, S, _NSAMPLE_LIST[2]), jnp.int32),
        ],
    )(nxp, xyzt)
    return outs


def _fold_params(layers):
    s = np.float32(np.sqrt(1.0 + _BN_EPS))
    As, cs = [], []
    prev_scale = None
    prev_shift = None
    for (W, b, g, be) in layers:
        if prev_scale is None:
            A = W
            c = b
        else:
            A = prev_scale[:, None] * W
            c = prev_shift @ W + b
        As.append(A)
        cs.append(c)
        prev_scale = g / s
        prev_shift = be
    return As, cs, prev_scale, prev_shift


def _sc_gather(table, idx2d, R):
    NC, NS = 2, 16
    NW = NC * NS
    rows_w = R // NW
    n_idxrows_w = rows_w // 128
    CH = 4
    n_outer = n_idxrows_w // CH
    mesh = plsc.VectorSubcoreMesh(core_axis_name="c", subcore_axis_name="s")

    @functools.partial(
        pl.kernel,
        mesh=mesh,
        out_type=jax.ShapeDtypeStruct((R, _TBL_W), jnp.float32),
        scratch_types=[
            pltpu.VMEM((CH, 128), jnp.int32),
            pltpu.VMEM((CH * 128, _TBL_W), jnp.float32),
            pltpu.SemaphoreType.DMA,
        ],
    )
    def k(table_hbm, idx_hbm, out_hbm, idx_v, rows_v, sem):
        wid = lax.axis_index("s") * NC + lax.axis_index("c")
        idxrow0 = wid * n_idxrows_w

        def body(j, carry):
            r0 = idxrow0 + j * CH
            pltpu.sync_copy(idx_hbm.at[pl.ds(r0, CH)], idx_v)
            cps = [
                pltpu.async_copy(
                    table_hbm.at[idx_v.at[i]],
                    rows_v.at[pl.ds(i * 128, 128)],
                    sem,
                )
                for i in range(CH)
            ]
            for cp in cps:
                cp.wait()
            pltpu.sync_copy(rows_v, out_hbm.at[pl.ds(r0 * 128, CH * 128)])
            return carry

        lax.fori_loop(0, n_outer, body, 0)

    return k(table, idx2d)


def _mlp_body(nblk, K, g_ref, nx_ref, w1x, w1, c1, w2, c2, w3, c3, gs, be, o_ref):
    x = g_ref[...]
    corr = jnp.dot(nx_ref[...], w1x[...], preferred_element_type=jnp.float32)
    h = jnp.dot(x, w1[...], preferred_element_type=jnp.float32) + c1[...]
    C1 = h.shape[-1]
    h = h.reshape(nblk, K, C1) - corr[:, None, :]
    h = jnp.maximum(h, 0.0).reshape(nblk * K, C1)
    h = jnp.maximum(jnp.dot(h, w2[...], preferred_element_type=jnp.float32) + c2[...], 0.0)
    h = jnp.maximum(jnp.dot(h, w3[...], preferred_element_type=jnp.float32) + c3[...], 0.0)
    h = h * gs[...] + be[...]
    C = h.shape[-1]
    h = h.reshape(nblk, K, C).max(axis=1)
    o_ref[...] = h


def _mlp_max(x, nxp, layers, K, nblk):
    R = nxp.shape[0]
    As, cs, gs, be = _fold_params(layers)
    A1 = jnp.zeros((_TBL_W, As[0].shape[1]), jnp.float32).at[: As[0].shape[0]].set(As[0])
    A1x = jnp.zeros((8, As[0].shape[1]), jnp.float32).at[:3].set(As[0][16:19])
    C1, C2, C3 = As[0].shape[1], As[1].shape[1], As[2].shape[1]
    grid = (R // nblk,)
    out = pl.pallas_call(
        functools.partial(_mlp_body, nblk, K),
        grid=grid,
        in_specs=[
            pl.BlockSpec((nblk * K, _TBL_W), lambda g: (g, 0)),
            pl.BlockSpec((nblk, 8), lambda g: (g, 0)),
            pl.BlockSpec((8, C1), lambda g: (0, 0)),
            pl.BlockSpec((_TBL_W, C1), lambda g: (0, 0)),
            pl.BlockSpec((1, C1), lambda g: (0, 0)),
            pl.BlockSpec((C1, C2), lambda g: (0, 0)),
            pl.BlockSpec((1, C2), lambda g: (0, 0)),
            pl.BlockSpec((C2, C3), lambda g: (0, 0)),
            pl.BlockSpec((1, C3), lambda g: (0, 0)),
            pl.BlockSpec((1, C3), lambda g: (0, 0)),
            pl.BlockSpec((1, C3), lambda g: (0, 0)),
        ],
        out_specs=pl.BlockSpec((nblk, C3), lambda g: (g, 0)),
        out_shape=jax.ShapeDtypeStruct((R, C3), jnp.float32),
    )(
        x,
        nxp,
        A1x,
        A1,
        cs[0][None, :],
        As[1],
        cs[1][None, :],
        As[2],
        cs[2][None, :],
        gs[None, :],
        be[None, :],
    )
    return out


def kernel(xyz, points, params):
    B, N, _ = xyz.shape
    S = _NPOINT
    fps_idx, new_xyz = _fps(xyz, S)

    feats = jnp.concatenate(
        [points, xyz, jnp.zeros((B, N, _TBL_W - 19), jnp.float32)], axis=-1
    ).reshape(B * N, _TBL_W)

    group_idx = _ball_query_all(xyz, new_xyz)
    boff = (jnp.arange(B, dtype=jnp.int32) * N)[:, None, None]
    flat = jnp.concatenate([(gi + boff).reshape(-1) for gi in group_idx], axis=0)
    R_all = flat.shape[0]
    gathered = _sc_gather(feats, flat.reshape(R_all // 128, 128), R_all)

    nxp = jnp.concatenate([new_xyz, jnp.zeros((B, S, 5), jnp.float32)], axis=-1)
    nxp = nxp.reshape(B * S, 8)
    outs = []
    off = 0
    for i, K in enumerate(_NSAMPLE_LIST):
        n = B * S * K
        out = _mlp_max(gathered[off : off + n], nxp, params[i], K, nblk=64)
        off += n
        outs.append(out.reshape(B, S, -1))
    return (new_xyz, jnp.concatenate(outs, axis=-1))


# speedup vs baseline: 1.1729x; 1.1729x over previous
import functools
import jax
import jax.numpy as jnp
import numpy as np
from jax import lax
from jax.experimental import pallas as pl
from jax.experimental.pallas import tpu as pltpu, tpu_sc as plsc

_NPOINT = 512
_RADIUS_LIST = [0.1, 0.2, 0.4]
_NSAMPLE_LIST = [16, 32, 128]
_BN_EPS = 1e-3
_CIN_PAD = 32
_TBL_W = 128


_FPS_R, _FPS_L = 64, 128


def _fps_pallas_body(B, x_ref, y_ref, z_ref, idx_ref, cx_ref, cy_ref, cz_ref):
    Xs = [x_ref[b] for b in range(B)]
    Ys = [y_ref[b] for b in range(B)]
    Zs = [z_ref[b] for b in range(B)]
    lin = (
        jax.lax.broadcasted_iota(jnp.int32, (_FPS_R, _FPS_L), 0) * _FPS_L
        + jax.lax.broadcasted_iota(jnp.int32, (_FPS_R, _FPS_L), 1)
    )
    lin_s = (
        jax.lax.broadcasted_iota(jnp.int32, (4, 128), 0) * 128
        + jax.lax.broadcasted_iota(jnp.int32, (4, 128), 1)
    )
    N = _FPS_R * _FPS_L

    def body(i, carry):
        dist, far, idxacc, cxa, cya, cza = carry
        out = ([], [], [], [], [], [])
        rec = lin_s == i
        for b in range(B):
            selm = lin == far[b]
            cx = jnp.sum(jnp.where(selm, Xs[b], 0.0), keepdims=True)
            cy = jnp.sum(jnp.where(selm, Ys[b], 0.0), keepdims=True)
            cz = jnp.sum(jnp.where(selm, Zs[b], 0.0), keepdims=True)
            d = (Xs[b] - cx) ** 2 + (Ys[b] - cy) ** 2 + (Zs[b] - cz) ** 2
            db = jnp.minimum(dist[b], d)
            m = jnp.max(db, keepdims=True)
            far2 = jnp.min(jnp.where(db == m, lin, N), keepdims=True)
            out[0].append(db)
            out[1].append(far2)
            out[2].append(jnp.where(rec, far[b], idxacc[b]))
            out[3].append(jnp.where(rec, cx, cxa[b]))
            out[4].append(jnp.where(rec, cy, cya[b]))
            out[5].append(jnp.where(rec, cz, cza[b]))
        return out

    init = (
        [jnp.full((_FPS_R, _FPS_L), 1e10, jnp.float32)] * B,
        [jnp.zeros((1, 1), jnp.int32)] * B,
        [jnp.zeros((4, 128), jnp.int32)] * B,
        [jnp.zeros((4, 128), jnp.float32)] * B,
        [jnp.zeros((4, 128), jnp.float32)] * B,
        [jnp.zeros((4, 128), jnp.float32)] * B,
    )
    _, _, idxacc, cxa, cya, cza = jax.lax.fori_loop(0, _NPOINT, body, init)
    for b in range(B):
        idx_ref[b] = idxacc[b]
        cx_ref[b] = cxa[b]
        cy_ref[b] = cya[b]
        cz_ref[b] = cza[b]


def _fps(xyz, npoint):
    B, N, _ = xyz.shape
    X = xyz[..., 0].reshape(B, _FPS_R, _FPS_L)
    Y = xyz[..., 1].reshape(B, _FPS_R, _FPS_L)
    Z = xyz[..., 2].reshape(B, _FPS_R, _FPS_L)
    blk = pl.BlockSpec((B, _FPS_R, _FPS_L), lambda: (0, 0, 0))
    oblk = pl.BlockSpec((B, 4, 128), lambda: (0, 0, 0))
    idx, cx, cy, cz = pl.pallas_call(
        functools.partial(_fps_pallas_body, B),
        grid=(),
        in_specs=[blk, blk, blk],
        out_specs=[oblk, oblk, oblk, oblk],
        out_shape=[
            jax.ShapeDtypeStruct((B, 4, 128), jnp.int32),
            jax.ShapeDtypeStruct((B, 4, 128), jnp.float32),
            jax.ShapeDtypeStruct((B, 4, 128), jnp.float32),
            jax.ShapeDtypeStruct((B, 4, 128), jnp.float32),
        ],
    )(X, Y, Z)
    fps_idx = idx.reshape(B, npoint)
    new_xyz = jnp.stack(
        [cx.reshape(B, npoint), cy.reshape(B, npoint), cz.reshape(B, npoint)], axis=-1
    )
    return fps_idx, new_xyz


_BQ_SBLK = 32


def _bq_body(nxp_ref, xyzt_ref, o1_ref, o2_ref, o3_ref):
    nxp = nxp_ref[0]
    xyzt = xyzt_ref[0]
    N = xyzt.shape[1]
    d2 = (
        jnp.sum(nxp * nxp, axis=1, keepdims=True)
        + jnp.sum(xyzt * xyzt, axis=0, keepdims=True)
        - 2.0 * jnp.dot(nxp, xyzt, preferred_element_type=jnp.float32)
    )
    linj = jax.lax.broadcasted_iota(jnp.int32, (_BQ_SBLK, N), 1).astype(jnp.float32)
    CW = 1024
    for radius, K, o_ref in (
        (_RADIUS_LIST[0], _NSAMPLE_LIST[0], o1_ref),
        (_RADIUS_LIST[1], _NSAMPLE_LIST[1], o2_ref),
        (_RADIUS_LIST[2], _NSAMPLE_LIST[2], o3_ref),
    ):
        c = jnp.where(d2 <= radius * radius, 1.0, 0.0)
        sh = 1
        while sh < N:
            c = c + jnp.concatenate(
                [jnp.zeros((_BQ_SBLK, sh), jnp.float32), c[:, : N - sh]], axis=1
            )
            sh *= 2
        cntK = jnp.sum(jnp.where(c <= jnp.float32(K - 1), 1.0, 0.0), axis=1)
        bound = jnp.max(cntK)
        kio = jax.lax.broadcasted_iota(jnp.int32, (_BQ_SBLK, K), 1)
        acc = jnp.full((_BQ_SBLK, K), jnp.float32(N), jnp.float32)

        for q in range(N // CW):
            cq = jax.lax.slice_in_dim(c, q * CW, (q + 1) * CW, axis=1)
            lq = jax.lax.slice_in_dim(linj, q * CW, (q + 1) * CW, axis=1)

            def _chunk(acc=acc, cq=cq, lq=lq):
                a = acc
                for k in range(K):
                    cand = jnp.where(cq == jnp.float32(k + 1), lq, jnp.float32(N))
                    idxk = jnp.min(cand, axis=1)
                    a = jnp.where(kio == k, jnp.minimum(a, idxk[:, None]), a)
                return a

            acc = jax.lax.cond(jnp.float32(q * CW) <= bound, _chunk, lambda acc=acc: acc)
        acc = acc.astype(jnp.int32)
        acc = jnp.where(acc == N, acc[:, :1], acc)
        o_ref[0] = acc


def _ball_query_all(xyz, new_xyz):
    B, N, _ = xyz.shape
    S = new_xyz.shape[1]
    nxp = jnp.concatenate([new_xyz, jnp.zeros((B, S, 5), jnp.float32)], axis=-1)
    xyzt = jnp.concatenate(
        [jnp.swapaxes(xyz, 1, 2), jnp.zeros((B, 5, N), jnp.float32)], axis=1
    )
    outs = pl.pallas_call(
        _bq_body,
        grid=(B, S // _BQ_SBLK),
        in_specs=[
            pl.BlockSpec((1, _BQ_SBLK, 8), lambda b, s: (b, s, 0)),
            pl.BlockSpec((1, 8, N), lambda b, s: (b, 0, 0)),
        ],
        out_specs=[
            pl.BlockSpec((1, _BQ_SBLK, _NSAMPLE_LIST[0]), lambda b, s: (b, s, 0)),
            pl.BlockSpec((1, _BQ_SBLK, _NSAMPLE_LIST[1]), lambda b, s: (b, s, 0)),
            pl.BlockSpec((1, _BQ_SBLK, _NSAMPLE_LIST[2]), lambda b, s: (b, s, 0)),
        ],
        out_shape=[
            jax.ShapeDtypeStruct((B, S, _NSAMPLE_LIST[0]), jnp.int32),
            jax.ShapeDtypeStruct((B, S, _NSAMPLE_LIST[1]), jnp.int32),
            jax.ShapeDtypeStruct((B, S, _NSAMPLE_LIST[2]), jnp.int32),
        ],
    )(nxp, xyzt)
    return outs


def _fold_params(layers):
    s = np.float32(np.sqrt(1.0 + _BN_EPS))
    As, cs = [], []
    prev_scale = None
    prev_shift = None
    for (W, b, g, be) in layers:
        if prev_scale is None:
            A = W
            c = b
        else:
            A = prev_scale[:, None] * W
            c = prev_shift @ W + b
        As.append(A)
        cs.append(c)
        prev_scale = g / s
        prev_shift = be
    return As, cs, prev_scale, prev_shift


def _sc_gather(table, idx2d, R):
    NC, NS = 2, 16
    NW = NC * NS
    rows_w = R // NW
    n_idxrows_w = rows_w // 128
    CH = 4
    n_outer = n_idxrows_w // CH
    mesh = plsc.VectorSubcoreMesh(core_axis_name="c", subcore_axis_name="s")

    @functools.partial(
        pl.kernel,
        mesh=mesh,
        out_type=jax.ShapeDtypeStruct((R, _TBL_W), jnp.float32),
        scratch_types=[
            pltpu.VMEM((CH, 128), jnp.int32),
            pltpu.VMEM((CH * 128, _TBL_W), jnp.float32),
            pltpu.SemaphoreType.DMA,
        ],
    )
    def k(table_hbm, idx_hbm, out_hbm, idx_v, rows_v, sem):
        wid = lax.axis_index("s") * NC + lax.axis_index("c")
        idxrow0 = wid * n_idxrows_w

        def body(j, carry):
            r0 = idxrow0 + j * CH
            pltpu.sync_copy(idx_hbm.at[pl.ds(r0, CH)], idx_v)
            cps = [
                pltpu.async_copy(
                    table_hbm.at[idx_v.at[i]],
                    rows_v.at[pl.ds(i * 128, 128)],
                    sem,
                )
                for i in range(CH)
            ]
            for cp in cps:
                cp.wait()
            pltpu.sync_copy(rows_v, out_hbm.at[pl.ds(r0 * 128, CH * 128)])
            return carry

        lax.fori_loop(0, n_outer, body, 0)

    return k(table, idx2d)


def _mlp_body(nblk, K, g_ref, nx_ref, w1x, w1, c1, w2, c2, w3, c3, gs, be, o_ref):
    x = g_ref[...]
    corr = jnp.dot(nx_ref[...], w1x[...], preferred_element_type=jnp.float32)
    h = jnp.dot(x, w1[...], preferred_element_type=jnp.float32) + c1[...]
    C1 = h.shape[-1]
    h = h.reshape(nblk, K, C1) - corr[:, None, :]
    h = jnp.maximum(h, 0.0).reshape(nblk * K, C1)
    h = jnp.maximum(jnp.dot(h, w2[...], preferred_element_type=jnp.float32) + c2[...], 0.0)
    h = jnp.maximum(jnp.dot(h, w3[...], preferred_element_type=jnp.float32) + c3[...], 0.0)
    h = h * gs[...] + be[...]
    C = h.shape[-1]
    h = h.reshape(nblk, K, C).max(axis=1)
    o_ref[...] = h


def _mlp_max(x, nxp, layers, K, nblk):
    R = nxp.shape[0]
    As, cs, gs, be = _fold_params(layers)
    A1 = jnp.zeros((_TBL_W, As[0].shape[1]), jnp.float32).at[: As[0].shape[0]].set(As[0])
    A1x = jnp.zeros((8, As[0].shape[1]), jnp.float32).at[:3].set(As[0][16:19])
    C1, C2, C3 = As[0].shape[1], As[1].shape[1], As[2].shape[1]
    grid = (R // nblk,)
    out = pl.pallas_call(
        functools.partial(_mlp_body, nblk, K),
        grid=grid,
        in_specs=[
            pl.BlockSpec((nblk * K, _TBL_W), lambda g: (g, 0)),
            pl.BlockSpec((nblk, 8), lambda g: (g, 0)),
            pl.BlockSpec((8, C1), lambda g: (0, 0)),
            pl.BlockSpec((_TBL_W, C1), lambda g: (0, 0)),
            pl.BlockSpec((1, C1), lambda g: (0, 0)),
            pl.BlockSpec((C1, C2), lambda g: (0, 0)),
            pl.BlockSpec((1, C2), lambda g: (0, 0)),
            pl.BlockSpec((C2, C3), lambda g: (0, 0)),
            pl.BlockSpec((1, C3), lambda g: (0, 0)),
            pl.BlockSpec((1, C3), lambda g: (0, 0)),
            pl.BlockSpec((1, C3), lambda g: (0, 0)),
        ],
        out_specs=pl.BlockSpec((nblk, C3), lambda g: (g, 0)),
        out_shape=jax.ShapeDtypeStruct((R, C3), jnp.float32),
    )(
        x,
        nxp,
        A1x,
        A1,
        cs[0][None, :],
        As[1],
        cs[1][None, :],
        As[2],
        cs[2][None, :],
        gs[None, :],
        be[None, :],
    )
    return out


def kernel(xyz, points, params):
    B, N, _ = xyz.shape
    S = _NPOINT
    fps_idx, new_xyz = _fps(xyz, S)

    feats = jnp.concatenate(
        [points, xyz, jnp.zeros((B, N, _TBL_W - 19), jnp.float32)], axis=-1
    ).reshape(B * N, _TBL_W)

    group_idx = _ball_query_all(xyz, new_xyz)
    boff = (jnp.arange(B, dtype=jnp.int32) * N)[:, None, None]
    flat = jnp.concatenate([(gi + boff).reshape(-1) for gi in group_idx], axis=0)
    R_all = flat.shape[0]
    gathered = _sc_gather(feats, flat.reshape(R_all // 128, 128), R_all)

    nxp = jnp.concatenate([new_xyz, jnp.zeros((B, S, 5), jnp.float32)], axis=-1)
    nxp = nxp.reshape(B * S, 8)
    outs = []
    off = 0
    for i, K in enumerate(_NSAMPLE_LIST):
        n = B * S * K
        out = _mlp_max(gathered[off : off + n], nxp, params[i], K, nblk=128)
        off += n
        outs.append(out.reshape(B, S, -1))
    return (new_xyz, jnp.concatenate(outs, axis=-1))


# speedup vs baseline: 1.1763x; 1.0030x over previous
import functools
import jax
import jax.numpy as jnp
import numpy as np
from jax import lax
from jax.experimental import pallas as pl
from jax.experimental.pallas import tpu as pltpu, tpu_sc as plsc

_NPOINT = 512
_RADIUS_LIST = [0.1, 0.2, 0.4]
_NSAMPLE_LIST = [16, 32, 128]
_BN_EPS = 1e-3
_CIN_PAD = 32
_TBL_W = 128


_FPS_R, _FPS_L = 64, 128


def _fps_pallas_body(B, x_ref, y_ref, z_ref, idx_ref, cx_ref, cy_ref, cz_ref):
    Xs = [x_ref[b] for b in range(B)]
    Ys = [y_ref[b] for b in range(B)]
    Zs = [z_ref[b] for b in range(B)]
    lin = (
        jax.lax.broadcasted_iota(jnp.int32, (_FPS_R, _FPS_L), 0) * _FPS_L
        + jax.lax.broadcasted_iota(jnp.int32, (_FPS_R, _FPS_L), 1)
    )
    lin_s = (
        jax.lax.broadcasted_iota(jnp.int32, (4, 128), 0) * 128
        + jax.lax.broadcasted_iota(jnp.int32, (4, 128), 1)
    )
    N = _FPS_R * _FPS_L

    def body(i, carry):
        dist, far, idxacc, cxa, cya, cza = carry
        out = ([], [], [], [], [], [])
        rec = lin_s == i
        for b in range(B):
            selm = lin == far[b]
            cx = jnp.sum(jnp.where(selm, Xs[b], 0.0), keepdims=True)
            cy = jnp.sum(jnp.where(selm, Ys[b], 0.0), keepdims=True)
            cz = jnp.sum(jnp.where(selm, Zs[b], 0.0), keepdims=True)
            d = (Xs[b] - cx) ** 2 + (Ys[b] - cy) ** 2 + (Zs[b] - cz) ** 2
            db = jnp.minimum(dist[b], d)
            m = jnp.max(db, keepdims=True)
            far2 = jnp.min(jnp.where(db == m, lin, N), keepdims=True)
            out[0].append(db)
            out[1].append(far2)
            out[2].append(jnp.where(rec, far[b], idxacc[b]))
            out[3].append(jnp.where(rec, cx, cxa[b]))
            out[4].append(jnp.where(rec, cy, cya[b]))
            out[5].append(jnp.where(rec, cz, cza[b]))
        return out

    init = (
        [jnp.full((_FPS_R, _FPS_L), 1e10, jnp.float32)] * B,
        [jnp.zeros((1, 1), jnp.int32)] * B,
        [jnp.zeros((4, 128), jnp.int32)] * B,
        [jnp.zeros((4, 128), jnp.float32)] * B,
        [jnp.zeros((4, 128), jnp.float32)] * B,
        [jnp.zeros((4, 128), jnp.float32)] * B,
    )
    _, _, idxacc, cxa, cya, cza = jax.lax.fori_loop(0, _NPOINT, body, init)
    for b in range(B):
        idx_ref[b] = idxacc[b]
        cx_ref[b] = cxa[b]
        cy_ref[b] = cya[b]
        cz_ref[b] = cza[b]


def _fps(xyz, npoint):
    B, N, _ = xyz.shape
    X = xyz[..., 0].reshape(B, _FPS_R, _FPS_L)
    Y = xyz[..., 1].reshape(B, _FPS_R, _FPS_L)
    Z = xyz[..., 2].reshape(B, _FPS_R, _FPS_L)
    blk = pl.BlockSpec((B, _FPS_R, _FPS_L), lambda: (0, 0, 0))
    oblk = pl.BlockSpec((B, 4, 128), lambda: (0, 0, 0))
    idx, cx, cy, cz = pl.pallas_call(
        functools.partial(_fps_pallas_body, B),
        grid=(),
        in_specs=[blk, blk, blk],
        out_specs=[oblk, oblk, oblk, oblk],
        out_shape=[
            jax.ShapeDtypeStruct((B, 4, 128), jnp.int32),
            jax.ShapeDtypeStruct((B, 4, 128), jnp.float32),
            jax.ShapeDtypeStruct((B, 4, 128), jnp.float32),
            jax.ShapeDtypeStruct((B, 4, 128), jnp.float32),
        ],
    )(X, Y, Z)
    fps_idx = idx.reshape(B, npoint)
    new_xyz = jnp.stack(
        [cx.reshape(B, npoint), cy.reshape(B, npoint), cz.reshape(B, npoint)], axis=-1
    )
    return fps_idx, new_xyz


_BQ_SBLK = 32


def _bq_body(nxp_ref, xyzt_ref, o1_ref, o2_ref, o3_ref):
    nxp = nxp_ref[0]
    xyzt = xyzt_ref[0]
    N = xyzt.shape[1]
    d2 = (
        jnp.sum(nxp * nxp, axis=1, keepdims=True)
        + jnp.sum(xyzt * xyzt, axis=0, keepdims=True)
        - 2.0 * jnp.dot(nxp, xyzt, preferred_element_type=jnp.float32)
    )
    linj = jax.lax.broadcasted_iota(jnp.int32, (_BQ_SBLK, N), 1).astype(jnp.float32)
    CW = 1024
    for radius, K, o_ref in (
        (_RADIUS_LIST[0], _NSAMPLE_LIST[0], o1_ref),
        (_RADIUS_LIST[1], _NSAMPLE_LIST[1], o2_ref),
        (_RADIUS_LIST[2], _NSAMPLE_LIST[2], o3_ref),
    ):
        c = jnp.where(d2 <= radius * radius, 1.0, 0.0)
        sh = 1
        while sh < N:
            c = c + jnp.concatenate(
                [jnp.zeros((_BQ_SBLK, sh), jnp.float32), c[:, : N - sh]], axis=1
            )
            sh *= 2
        cntK = jnp.sum(jnp.where(c <= jnp.float32(K - 1), 1.0, 0.0), axis=1)
        bound = jnp.max(cntK)
        kio = jax.lax.broadcasted_iota(jnp.int32, (_BQ_SBLK, K), 1)
        acc = jnp.full((_BQ_SBLK, K), jnp.float32(N), jnp.float32)

        for q in range(N // CW):
            cq = jax.lax.slice_in_dim(c, q * CW, (q + 1) * CW, axis=1)
            lq = jax.lax.slice_in_dim(linj, q * CW, (q + 1) * CW, axis=1)

            def _chunk(acc=acc, cq=cq, lq=lq):
                a = acc
                for k in range(K):
                    cand = jnp.where(cq == jnp.float32(k + 1), lq, jnp.float32(N))
                    idxk = jnp.min(cand, axis=1)
                    a = jnp.where(kio == k, jnp.minimum(a, idxk[:, None]), a)
                return a

            acc = jax.lax.cond(jnp.float32(q * CW) <= bound, _chunk, lambda acc=acc: acc)
        acc = acc.astype(jnp.int32)
        acc = jnp.where(acc == N, acc[:, :1], acc)
        o_ref[0] = acc


def _ball_query_all(xyz, new_xyz):
    B, N, _ = xyz.shape
    S = new_xyz.shape[1]
    nxp = jnp.concatenate([new_xyz, jnp.zeros((B, S, 5), jnp.float32)], axis=-1)
    xyzt = jnp.concatenate(
        [jnp.swapaxes(xyz, 1, 2), jnp.zeros((B, 5, N), jnp.float32)], axis=1
    )
    outs = pl.pallas_call(
        _bq_body,
        grid=(B, S // _BQ_SBLK),
        in_specs=[
            pl.BlockSpec((1, _BQ_SBLK, 8), lambda b, s: (b, s, 0)),
            pl.BlockSpec((1, 8, N), lambda b, s: (b, 0, 0)),
        ],
        out_specs=[
            pl.BlockSpec((1, _BQ_SBLK, _NSAMPLE_LIST[0]), lambda b, s: (b, s, 0)),
            pl.BlockSpec((1, _BQ_SBLK, _NSAMPLE_LIST[1]), lambda b, s: (b, s, 0)),
            pl.BlockSpec((1, _BQ_SBLK, _NSAMPLE_LIST[2]), lambda b, s: (b, s, 0)),
        ],
        out_shape=[
            jax.ShapeDtypeStruct((B, S, _NSAMPLE_LIST[0]), jnp.int32),
            jax.ShapeDtypeStruct((B, S, _NSAMPLE_LIST[1]), jnp.int32),
            jax.ShapeDtypeStruct((B, S, _NSAMPLE_LIST[2]), jnp.int32),
        ],
    )(nxp, xyzt)
    return outs


def _fold_params(layers):
    s = np.float32(np.sqrt(1.0 + _BN_EPS))
    As, cs = [], []
    prev_scale = None
    prev_shift = None
    for (W, b, g, be) in layers:
        if prev_scale is None:
            A = W
            c = b
        else:
            A = prev_scale[:, None] * W
            c = prev_shift @ W + b
        As.append(A)
        cs.append(c)
        prev_scale = g / s
        prev_shift = be
    return As, cs, prev_scale, prev_shift


def _sc_gather(table, idx2d, R):
    NC, NS = 2, 16
    NW = NC * NS
    rows_w = R // NW
    n_idxrows_w = rows_w // 128
    CH = 4
    n_outer = n_idxrows_w // CH
    mesh = plsc.VectorSubcoreMesh(core_axis_name="c", subcore_axis_name="s")

    @functools.partial(
        pl.kernel,
        mesh=mesh,
        out_type=jax.ShapeDtypeStruct((R, _TBL_W), jnp.float32),
        scratch_types=[
            pltpu.VMEM((CH, 128), jnp.int32),
            pltpu.VMEM((CH * 128, _TBL_W), jnp.float32),
            pltpu.SemaphoreType.DMA,
        ],
    )
    def k(table_hbm, idx_hbm, out_hbm, idx_v, rows_v, sem):
        wid = lax.axis_index("s") * NC + lax.axis_index("c")
        idxrow0 = wid * n_idxrows_w

        def body(j, carry):
            r0 = idxrow0 + j * CH
            pltpu.sync_copy(idx_hbm.at[pl.ds(r0, CH)], idx_v)
            cps = [
                pltpu.async_copy(
                    table_hbm.at[idx_v.at[i]],
                    rows_v.at[pl.ds(i * 128, 128)],
                    sem,
                )
                for i in range(CH)
            ]
            for cp in cps:
                cp.wait()
            pltpu.sync_copy(rows_v, out_hbm.at[pl.ds(r0 * 128, CH * 128)])
            return carry

        lax.fori_loop(0, n_outer, body, 0)

    return k(table, idx2d)


def _mlp_body(nblk, K, g_ref, nx_ref, w1x, w1, c1, w2, c2, w3, c3, gs, be, o_ref):
    x = g_ref[...]
    corr = jnp.dot(nx_ref[...], w1x[...], preferred_element_type=jnp.float32)
    h = jnp.dot(x, w1[...], preferred_element_type=jnp.float32) + c1[...]
    C1 = h.shape[-1]
    h = h.reshape(nblk, K, C1) - corr[:, None, :]
    h = jnp.maximum(h, 0.0).reshape(nblk * K, C1)
    h = jnp.maximum(jnp.dot(h, w2[...], preferred_element_type=jnp.float32) + c2[...], 0.0)
    h = jnp.maximum(jnp.dot(h, w3[...], preferred_element_type=jnp.float32) + c3[...], 0.0)
    h = h * gs[...] + be[...]
    C = h.shape[-1]
    h = h.reshape(nblk, K, C).max(axis=1)
    o_ref[...] = h


def _mlp_max(x, nxp, layers, K, nblk):
    R = nxp.shape[0]
    As, cs, gs, be = _fold_params(layers)
    A1 = jnp.zeros((_TBL_W, As[0].shape[1]), jnp.float32).at[: As[0].shape[0]].set(As[0])
    A1x = jnp.zeros((8, As[0].shape[1]), jnp.float32).at[:3].set(As[0][16:19])
    C1, C2, C3 = As[0].shape[1], As[1].shape[1], As[2].shape[1]
    grid = (R // nblk,)
    out = pl.pallas_call(
        functools.partial(_mlp_body, nblk, K),
        grid=grid,
        in_specs=[
            pl.BlockSpec((nblk * K, _TBL_W), lambda g: (g, 0)),
            pl.BlockSpec((nblk, 8), lambda g: (g, 0)),
            pl.BlockSpec((8, C1), lambda g: (0, 0)),
            pl.BlockSpec((_TBL_W, C1), lambda g: (0, 0)),
            pl.BlockSpec((1, C1), lambda g: (0, 0)),
            pl.BlockSpec((C1, C2), lambda g: (0, 0)),
            pl.BlockSpec((1, C2), lambda g: (0, 0)),
            pl.BlockSpec((C2, C3), lambda g: (0, 0)),
            pl.BlockSpec((1, C3), lambda g: (0, 0)),
            pl.BlockSpec((1, C3), lambda g: (0, 0)),
            pl.BlockSpec((1, C3), lambda g: (0, 0)),
        ],
        out_specs=pl.BlockSpec((nblk, C3), lambda g: (g, 0)),
        out_shape=jax.ShapeDtypeStruct((R, C3), jnp.float32),
    )(
        x,
        nxp,
        A1x,
        A1,
        cs[0][None, :],
        As[1],
        cs[1][None, :],
        As[2],
        cs[2][None, :],
        gs[None, :],
        be[None, :],
    )
    return out


def kernel(xyz, points, params):
    B, N, _ = xyz.shape
    S = _NPOINT
    fps_idx, new_xyz = _fps(xyz, S)

    feats = jnp.concatenate(
        [points, xyz, jnp.zeros((B, N, _TBL_W - 19), jnp.float32)], axis=-1
    ).reshape(B * N, _TBL_W)

    group_idx = _ball_query_all(xyz, new_xyz)
    boff = (jnp.arange(B, dtype=jnp.int32) * N)[:, None, None]
    flat = jnp.concatenate([(gi + boff).reshape(-1) for gi in group_idx], axis=0)
    R_all = flat.shape[0]
    gathered = _sc_gather(feats, flat.reshape(R_all // 128, 128), R_all)

    nxp = jnp.concatenate([new_xyz, jnp.zeros((B, S, 5), jnp.float32)], axis=-1)
    nxp = nxp.reshape(B * S, 8)
    outs = []
    off = 0
    for i, K in enumerate(_NSAMPLE_LIST):
        n = B * S * K
        out = _mlp_max(gathered[off : off + n], nxp, params[i], K, nblk=256)
        off += n
        outs.append(out.reshape(B, S, -1))
    return (new_xyz, jnp.concatenate(outs, axis=-1))


# speedup vs baseline: 1.1818x; 1.0046x over previous
import functools
import jax
import jax.numpy as jnp
import numpy as np
from jax import lax
from jax.experimental import pallas as pl
from jax.experimental.pallas import tpu as pltpu, tpu_sc as plsc

_NPOINT = 512
_RADIUS_LIST = [0.1, 0.2, 0.4]
_NSAMPLE_LIST = [16, 32, 128]
_BN_EPS = 1e-3
_CIN_PAD = 32
_TBL_W = 128


_FPS_R, _FPS_L = 64, 128


def _fps_pallas_body(B, x_ref, y_ref, z_ref, idx_ref, cx_ref, cy_ref, cz_ref):
    Xs = [x_ref[b] for b in range(B)]
    Ys = [y_ref[b] for b in range(B)]
    Zs = [z_ref[b] for b in range(B)]
    lin = (
        jax.lax.broadcasted_iota(jnp.int32, (_FPS_R, _FPS_L), 0) * _FPS_L
        + jax.lax.broadcasted_iota(jnp.int32, (_FPS_R, _FPS_L), 1)
    )
    lin_s = (
        jax.lax.broadcasted_iota(jnp.int32, (4, 128), 0) * 128
        + jax.lax.broadcasted_iota(jnp.int32, (4, 128), 1)
    )
    N = _FPS_R * _FPS_L

    def body(i, carry):
        dist, far, idxacc, cxa, cya, cza = carry
        out = ([], [], [], [], [], [])
        rec = lin_s == i
        for b in range(B):
            selm = lin == far[b]
            cx = jnp.sum(jnp.where(selm, Xs[b], 0.0), keepdims=True)
            cy = jnp.sum(jnp.where(selm, Ys[b], 0.0), keepdims=True)
            cz = jnp.sum(jnp.where(selm, Zs[b], 0.0), keepdims=True)
            d = (Xs[b] - cx) ** 2 + (Ys[b] - cy) ** 2 + (Zs[b] - cz) ** 2
            db = jnp.minimum(dist[b], d)
            m = jnp.max(db, keepdims=True)
            far2 = jnp.min(jnp.where(db == m, lin, N), keepdims=True)
            out[0].append(db)
            out[1].append(far2)
            out[2].append(jnp.where(rec, far[b], idxacc[b]))
            out[3].append(jnp.where(rec, cx, cxa[b]))
            out[4].append(jnp.where(rec, cy, cya[b]))
            out[5].append(jnp.where(rec, cz, cza[b]))
        return out

    init = (
        [jnp.full((_FPS_R, _FPS_L), 1e10, jnp.float32)] * B,
        [jnp.zeros((1, 1), jnp.int32)] * B,
        [jnp.zeros((4, 128), jnp.int32)] * B,
        [jnp.zeros((4, 128), jnp.float32)] * B,
        [jnp.zeros((4, 128), jnp.float32)] * B,
        [jnp.zeros((4, 128), jnp.float32)] * B,
    )
    _, _, idxacc, cxa, cya, cza = jax.lax.fori_loop(0, _NPOINT, body, init)
    for b in range(B):
        idx_ref[b] = idxacc[b]
        cx_ref[b] = cxa[b]
        cy_ref[b] = cya[b]
        cz_ref[b] = cza[b]


def _fps(xyz, npoint):
    B, N, _ = xyz.shape
    X = xyz[..., 0].reshape(B, _FPS_R, _FPS_L)
    Y = xyz[..., 1].reshape(B, _FPS_R, _FPS_L)
    Z = xyz[..., 2].reshape(B, _FPS_R, _FPS_L)
    blk = pl.BlockSpec((B, _FPS_R, _FPS_L), lambda: (0, 0, 0))
    oblk = pl.BlockSpec((B, 4, 128), lambda: (0, 0, 0))
    idx, cx, cy, cz = pl.pallas_call(
        functools.partial(_fps_pallas_body, B),
        grid=(),
        in_specs=[blk, blk, blk],
        out_specs=[oblk, oblk, oblk, oblk],
        out_shape=[
            jax.ShapeDtypeStruct((B, 4, 128), jnp.int32),
            jax.ShapeDtypeStruct((B, 4, 128), jnp.float32),
            jax.ShapeDtypeStruct((B, 4, 128), jnp.float32),
            jax.ShapeDtypeStruct((B, 4, 128), jnp.float32),
        ],
    )(X, Y, Z)
    fps_idx = idx.reshape(B, npoint)
    new_xyz = jnp.stack(
        [cx.reshape(B, npoint), cy.reshape(B, npoint), cz.reshape(B, npoint)], axis=-1
    )
    return fps_idx, new_xyz


_BQ_SBLK = 32


def _bq_body(nxp_ref, xyzt_ref, o1_ref, o2_ref, o3_ref):
    nxp = nxp_ref[0]
    xyzt = xyzt_ref[0]
    N = xyzt.shape[1]
    d2 = (
        jnp.sum(nxp * nxp, axis=1, keepdims=True)
        + jnp.sum(xyzt * xyzt, axis=0, keepdims=True)
        - 2.0 * jnp.dot(nxp, xyzt, preferred_element_type=jnp.float32)
    )
    linj = jax.lax.broadcasted_iota(jnp.int32, (_BQ_SBLK, N), 1).astype(jnp.float32)
    CW = 1024
    for radius, K, o_ref in (
        (_RADIUS_LIST[0], _NSAMPLE_LIST[0], o1_ref),
        (_RADIUS_LIST[1], _NSAMPLE_LIST[1], o2_ref),
        (_RADIUS_LIST[2], _NSAMPLE_LIST[2], o3_ref),
    ):
        c = jnp.where(d2 <= radius * radius, 1.0, 0.0)
        sh = 1
        while sh < N:
            c = c + jnp.concatenate(
                [jnp.zeros((_BQ_SBLK, sh), jnp.float32), c[:, : N - sh]], axis=1
            )
            sh *= 2
        cntK = jnp.sum(jnp.where(c <= jnp.float32(K - 1), 1.0, 0.0), axis=1)
        bound = jnp.max(cntK)
        kio = jax.lax.broadcasted_iota(jnp.int32, (_BQ_SBLK, K), 1)
        acc = jnp.full((_BQ_SBLK, K), jnp.float32(N), jnp.float32)

        for q in range(N // CW):
            cq = jax.lax.slice_in_dim(c, q * CW, (q + 1) * CW, axis=1)
            lq = jax.lax.slice_in_dim(linj, q * CW, (q + 1) * CW, axis=1)

            def _chunk(acc=acc, cq=cq, lq=lq):
                a = acc
                for k in range(K):
                    cand = jnp.where(cq == jnp.float32(k + 1), lq, jnp.float32(N))
                    idxk = jnp.min(cand, axis=1)
                    a = jnp.where(kio == k, jnp.minimum(a, idxk[:, None]), a)
                return a

            acc = jax.lax.cond(jnp.float32(q * CW) <= bound, _chunk, lambda acc=acc: acc)
        acc = acc.astype(jnp.int32)
        acc = jnp.where(acc == N, acc[:, :1], acc)
        o_ref[0] = acc


def _ball_query_all(xyz, new_xyz):
    B, N, _ = xyz.shape
    S = new_xyz.shape[1]
    nxp = jnp.concatenate([new_xyz, jnp.zeros((B, S, 5), jnp.float32)], axis=-1)
    xyzt = jnp.concatenate(
        [jnp.swapaxes(xyz, 1, 2), jnp.zeros((B, 5, N), jnp.float32)], axis=1
    )
    outs = pl.pallas_call(
        _bq_body,
        grid=(B, S // _BQ_SBLK),
        in_specs=[
            pl.BlockSpec((1, _BQ_SBLK, 8), lambda b, s: (b, s, 0)),
            pl.BlockSpec((1, 8, N), lambda b, s: (b, 0, 0)),
        ],
        out_specs=[
            pl.BlockSpec((1, _BQ_SBLK, _NSAMPLE_LIST[0]), lambda b, s: (b, s, 0)),
            pl.BlockSpec((1, _BQ_SBLK, _NSAMPLE_LIST[1]), lambda b, s: (b, s, 0)),
            pl.BlockSpec((1, _BQ_SBLK, _NSAMPLE_LIST[2]), lambda b, s: (b, s, 0)),
        ],
        out_shape=[
            jax.ShapeDtypeStruct((B, S, _NSAMPLE_LIST[0]), jnp.int32),
            jax.ShapeDtypeStruct((B, S, _NSAMPLE_LIST[1]), jnp.int32),
            jax.ShapeDtypeStruct((B, S, _NSAMPLE_LIST[2]), jnp.int32),
        ],
    )(nxp, xyzt)
    return outs


def _fold_params(layers):
    s = np.float32(np.sqrt(1.0 + _BN_EPS))
    As, cs = [], []
    prev_scale = None
    prev_shift = None
    for (W, b, g, be) in layers:
        if prev_scale is None:
            A = W
            c = b
        else:
            A = prev_scale[:, None] * W
            c = prev_shift @ W + b
        As.append(A)
        cs.append(c)
        prev_scale = g / s
        prev_shift = be
    return As, cs, prev_scale, prev_shift


def _sc_gather(table, idx2d, R):
    NC, NS = 2, 16
    NW = NC * NS
    rows_w = R // NW
    n_idxrows_w = rows_w // 128
    CH = 2
    n_outer = n_idxrows_w // (2 * CH)
    mesh = plsc.VectorSubcoreMesh(core_axis_name="c", subcore_axis_name="s")

    @functools.partial(
        pl.kernel,
        mesh=mesh,
        out_type=jax.ShapeDtypeStruct((R, _TBL_W), jnp.float32),
        scratch_types=[
            pltpu.VMEM((CH, 128), jnp.int32),
            pltpu.VMEM((CH, 128), jnp.int32),
            pltpu.VMEM((CH * 128, _TBL_W), jnp.float32),
            pltpu.VMEM((CH * 128, _TBL_W), jnp.float32),
            pltpu.SemaphoreType.DMA,
            pltpu.SemaphoreType.DMA,
            pltpu.SemaphoreType.DMA,
            pltpu.SemaphoreType.DMA,
        ],
    )
    def k(table_hbm, idx_hbm, out_hbm, idx0, idx1, rows0, rows1, sg0, sg1, so0, so1):
        wid = lax.axis_index("s") * NC + lax.axis_index("c")
        idxrow0 = wid * n_idxrows_w

        def body(j, carry):
            r0 = idxrow0 + j * 2 * CH
            r1 = r0 + CH
            pltpu.sync_copy(idx_hbm.at[pl.ds(r0, CH)], idx0)
            g0 = [
                pltpu.async_copy(
                    table_hbm.at[idx0.at[i]], rows0.at[pl.ds(i * 128, 128)], sg0
                )
                for i in range(CH)
            ]
            pltpu.sync_copy(idx_hbm.at[pl.ds(r1, CH)], idx1)
            g1 = [
                pltpu.async_copy(
                    table_hbm.at[idx1.at[i]], rows1.at[pl.ds(i * 128, 128)], sg1
                )
                for i in range(CH)
            ]
            for cp in g0:
                cp.wait()
            o0 = pltpu.async_copy(rows0, out_hbm.at[pl.ds(r0 * 128, CH * 128)], so0)
            for cp in g1:
                cp.wait()
            o1 = pltpu.async_copy(rows1, out_hbm.at[pl.ds(r1 * 128, CH * 128)], so1)
            o0.wait()
            o1.wait()
            return carry

        lax.fori_loop(0, n_outer, body, 0)

    return k(table, idx2d)


def _mlp_body(nblk, K, g_ref, nx_ref, w1x, w1, c1, w2, c2, w3, c3, gs, be, o_ref):
    x = g_ref[...]
    corr = jnp.dot(nx_ref[...], w1x[...], preferred_element_type=jnp.float32)
    h = jnp.dot(x, w1[...], preferred_element_type=jnp.float32) + c1[...]
    C1 = h.shape[-1]
    h = h.reshape(nblk, K, C1) - corr[:, None, :]
    h = jnp.maximum(h, 0.0).reshape(nblk * K, C1)
    h = jnp.maximum(jnp.dot(h, w2[...], preferred_element_type=jnp.float32) + c2[...], 0.0)
    h = jnp.maximum(jnp.dot(h, w3[...], preferred_element_type=jnp.float32) + c3[...], 0.0)
    h = h * gs[...] + be[...]
    C = h.shape[-1]
    h = h.reshape(nblk, K, C).max(axis=1)
    o_ref[...] = h


def _mlp_max(x, nxp, layers, K, nblk):
    R = nxp.shape[0]
    As, cs, gs, be = _fold_params(layers)
    A1 = jnp.zeros((_TBL_W, As[0].shape[1]), jnp.float32).at[: As[0].shape[0]].set(As[0])
    A1x = jnp.zeros((8, As[0].shape[1]), jnp.float32).at[:3].set(As[0][16:19])
    C1, C2, C3 = As[0].shape[1], As[1].shape[1], As[2].shape[1]
    grid = (R // nblk,)
    out = pl.pallas_call(
        functools.partial(_mlp_body, nblk, K),
        grid=grid,
        in_specs=[
            pl.BlockSpec((nblk * K, _TBL_W), lambda g: (g, 0)),
            pl.BlockSpec((nblk, 8), lambda g: (g, 0)),
            pl.BlockSpec((8, C1), lambda g: (0, 0)),
            pl.BlockSpec((_TBL_W, C1), lambda g: (0, 0)),
            pl.BlockSpec((1, C1), lambda g: (0, 0)),
            pl.BlockSpec((C1, C2), lambda g: (0, 0)),
            pl.BlockSpec((1, C2), lambda g: (0, 0)),
            pl.BlockSpec((C2, C3), lambda g: (0, 0)),
            pl.BlockSpec((1, C3), lambda g: (0, 0)),
            pl.BlockSpec((1, C3), lambda g: (0, 0)),
            pl.BlockSpec((1, C3), lambda g: (0, 0)),
        ],
        out_specs=pl.BlockSpec((nblk, C3), lambda g: (g, 0)),
        out_shape=jax.ShapeDtypeStruct((R, C3), jnp.float32),
    )(
        x,
        nxp,
        A1x,
        A1,
        cs[0][None, :],
        As[1],
        cs[1][None, :],
        As[2],
        cs[2][None, :],
        gs[None, :],
        be[None, :],
    )
    return out


def kernel(xyz, points, params):
    B, N, _ = xyz.shape
    S = _NPOINT
    fps_idx, new_xyz = _fps(xyz, S)

    feats = jnp.concatenate(
        [points, xyz, jnp.zeros((B, N, _TBL_W - 19), jnp.float32)], axis=-1
    ).reshape(B * N, _TBL_W)

    group_idx = _ball_query_all(xyz, new_xyz)
    boff = (jnp.arange(B, dtype=jnp.int32) * N)[:, None, None]
    flat = jnp.concatenate([(gi + boff).reshape(-1) for gi in group_idx], axis=0)
    R_all = flat.shape[0]
    gathered = _sc_gather(feats, flat.reshape(R_all // 128, 128), R_all)

    nxp = jnp.concatenate([new_xyz, jnp.zeros((B, S, 5), jnp.float32)], axis=-1)
    nxp = nxp.reshape(B * S, 8)
    outs = []
    off = 0
    for i, K in enumerate(_NSAMPLE_LIST):
        n = B * S * K
        out = _mlp_max(gathered[off : off + n], nxp, params[i], K, nblk=256)
        off += n
        outs.append(out.reshape(B, S, -1))
    return (new_xyz, jnp.concatenate(outs, axis=-1))
